# baseline (speedup 1.0000x reference)
import jax
import jax.numpy as jnp
from jax.experimental import pallas as pl
from jax.experimental.pallas import tpu as pltpu

_NSLOT = 4


def _body(tab_ref, out_ref, eh_ref, ew_ref, buf_ref, sem_ref):
    b = pl.program_id(0)
    ib = pl.program_id(1)
    nib = pl.num_programs(1)
    _, BI, H, W, D = buf_ref.shape
    s = b * nib + ib
    slot = jax.lax.rem(s, _NSLOT)

    @pl.when(s == 0)
    def _init():
        for i in range(H):
            eh_ref[i] = tab_ref[pl.ds(H - 1 - i, H), :]
        for j in range(W):
            ew_ref[j] = tab_ref[pl.ds(3 * W - 2 - j, W), :]

    def _copy(q, bb, iib):
        return pltpu.make_async_copy(
            buf_ref.at[q],
            out_ref.at[bb, pl.ds(iib * BI, BI)],
            sem_ref.at[q],
        )

    @pl.when(s >= _NSLOT)
    def _drain_slot():
        _copy(slot, b, ib).wait()

    eh = eh_ref[pl.ds(ib * BI, BI)]
    ew = ew_ref[...]
    buf_ref[slot] = eh[:, :, None, :] + ew[None, :, :, :]
    _copy(slot, b, ib).start()

    @pl.when(s == pl.num_programs(0) * nib - 1)
    def _drain_all():
        for q in range(_NSLOT):
            _copy(q, b, ib).wait()


def kernel(x, Wh, Ww):
    B, C, H, W = x.shape
    D = Wh.shape[1]
    BI = 8
    tab = jnp.concatenate([Ww, Wh], axis=0)[::-1]
    return pl.pallas_call(
        _body,
        grid=(B, H // BI),
        in_specs=[
            pl.BlockSpec((2 * (H + W) - 2, D), lambda b, ib: (0, 0)),
        ],
        out_specs=pl.BlockSpec(memory_space=pl.ANY),
        out_shape=jax.ShapeDtypeStruct((B, H, H, W, D), jnp.float32),
        scratch_shapes=[
            pltpu.VMEM((H, H, D), jnp.float32),
            pltpu.VMEM((W, W, D), jnp.float32),
            pltpu.VMEM((_NSLOT, BI, H, W, D), jnp.float32),
            pltpu.SemaphoreType.DMA((_NSLOT,)),
        ],
    )(tab)


# speedup vs baseline: 1.0239x; 1.0239x over previous
import jax
import jax.numpy as jnp
from jax.experimental import pallas as pl
from jax.experimental.pallas import tpu as pltpu


def _body(tab_ref, out_ref, eh_ref, ew_ref):
    b = pl.program_id(0)
    ib = pl.program_id(1)
    _, BI, H, W, D = out_ref.shape

    @pl.when(jnp.logical_and(b == 0, ib == 0))
    def _init():
        for i in range(H):
            eh_ref[i] = tab_ref[pl.ds(H - 1 - i, H), :]
        for j in range(W):
            ew_ref[j] = tab_ref[pl.ds(3 * W - 2 - j, W), :]

    eh = eh_ref[pl.ds(ib * BI, BI)]
    ew = ew_ref[...]
    out_ref[0] = eh[:, :, None, :] + ew[None, :, :, :]


def kernel(x, Wh, Ww):
    B, C, H, W = x.shape
    D = Wh.shape[1]
    BI = 8
    tab = jnp.concatenate([Ww, Wh], axis=0)[::-1]
    return pl.pallas_call(
        _body,
        grid=(B, H // BI),
        in_specs=[
            pl.BlockSpec((2 * (H + W) - 2, D), lambda b, ib: (0, 0)),
        ],
        out_specs=pl.BlockSpec((1, BI, H, W, D), lambda b, ib: (b, ib, 0, 0, 0)),
        out_shape=jax.ShapeDtypeStruct((B, H, H, W, D), jnp.float32),
        scratch_shapes=[
            pltpu.VMEM((H, H, D), jnp.float32),
            pltpu.VMEM((W, W, D), jnp.float32),
        ],
    )(tab)
